# Initial kernel scaffold; baseline (speedup 1.0000x reference)
import functools

import jax
import jax.numpy as jnp
from jax import lax
from jax.experimental import pallas as pl
from jax.experimental.pallas import tpu as pltpu
from jax.experimental.pallas import tpu_sc as plsc

N = 10000
E = 160000
D_IN = 256
D_HID = 256
N_CLASS = 64

NC = 2
NS = 16
CHUNK = 128
CPT = 40
EPAD = NC * NS * CPT * CHUNK
EROWS = EPAD // CHUNK
NPT = N // NS
WB = ((0, 128), (128, 128), (256, 128), (384, 128), (512, 113))

_mesh = lambda: plsc.VectorSubcoreMesh(core_axis_name="c", subcore_axis_name="s")


def _splat(v16, lane):
    idx = jnp.full((16,), lane, dtype=jnp.int32)
    return v16.at[idx].get(mode="promise_in_bounds")


def _deg_body(dst_hbm, ew_hbm, zcol_hbm, out_hbm, dst_buf, ew_buf, stage, acc_sh):
    c = lax.axis_index("c")
    s = lax.axis_index("s")
    base = s * NPT
    for off, sz in WB:
        pltpu.sync_copy(zcol_hbm.at[pl.ds(0, sz)], acc_sh.at[pl.ds(base + off, sz)])
    plsc.subcore_barrier()

    row0 = (c * NS + s) * CPT
    pltpu.sync_copy(dst_hbm.at[pl.ds(row0, CPT)], dst_buf)
    pltpu.sync_copy(ew_hbm.at[pl.ds(row0, CPT)], ew_buf)

    def body(j, carry):
        pltpu.sync_copy(ew_buf.at[j], acc_sh.at[dst_buf.at[j]], add=True)
        return carry

    lax.fori_loop(0, CPT, body, 0)
    plsc.subcore_barrier()
    for off, sz in WB:
        pltpu.sync_copy(acc_sh.at[pl.ds(base + off, sz)], stage.at[pl.ds(0, sz)])
        pltpu.sync_copy(stage.at[pl.ds(0, sz)], out_hbm.at[c, pl.ds(base + off, sz)])


def _deg_call(dst2d, ew3d, zcol):
    return pl.kernel(
        _deg_body,
        out_type=jax.ShapeDtypeStruct((NC, N, 1), jnp.float32),
        mesh=_mesh(),
        scratch_types=[
            pltpu.VMEM((CPT, CHUNK), jnp.int32),
            pltpu.VMEM((CPT, CHUNK, 1), jnp.float32),
            pltpu.VMEM((CHUNK, 1), jnp.float32),
            pltpu.VMEM_SHARED((N, 1), jnp.float32),
        ],
    )(dst2d, ew3d, zcol)


def _prop_body(dh, ta, tb, src_hbm, dst_hbm, ew_hbm, z_hbm, out_hbm,
               src_buf, dst_buf, ew_buf, rows0, rows1, acc_sh, sem0, sem1):
    c = lax.axis_index("c")
    s = lax.axis_index("s")
    base = s * NPT
    for off, sz in WB:
        pltpu.sync_copy(z_hbm.at[pl.ds(0, sz)], acc_sh.at[pl.ds(base + off, sz)])
    plsc.subcore_barrier()

    row0 = (c * NS + s) * CPT
    pltpu.sync_copy(src_hbm.at[pl.ds(row0, CPT)], src_buf)
    pltpu.sync_copy(dst_hbm.at[pl.ds(row0, CPT)], dst_buf)
    pltpu.sync_copy(ew_hbm.at[pl.ds(row0, CPT)], ew_buf)

    def scale(j, rows):
        def grp(gi, carry):
            w16 = ew_buf[j, pl.ds(gi * 16, 16)]
            for l in range(16):
                wl = _splat(w16, l)
                e = gi * 16 + l
                for v in range(dh // 16):
                    sl = pl.ds(v * 16, 16)
                    rows[e, sl] = rows[e, sl] * wl
            return carry
        lax.fori_loop(0, CHUNK // 16, grp, 0)

    def run(tbl):
        pltpu.async_copy(tbl.at[src_buf.at[0]], rows0, sem0)
        pltpu.async_copy(tbl.at[src_buf.at[1]], rows1, sem1)

        def step(j, rows, sem):
            pltpu.make_async_copy(tbl.at[src_buf.at[j]], rows, sem).wait()
            scale(j, rows)
            pltpu.sync_copy(rows, acc_sh.at[dst_buf.at[j]], add=True)
            @pl.when(j + 2 < CPT)
            def _():
                pltpu.async_copy(tbl.at[src_buf.at[j + 2]], rows, sem)

        def pair(g, carry):
            step(2 * g, rows0, sem0)
            step(2 * g + 1, rows1, sem1)
            return carry

        lax.fori_loop(0, CPT // 2, pair, 0)

    @pl.when(c == 0)
    def _():
        run(ta)

    @pl.when(c == 1)
    def _():
        run(tb)

    plsc.subcore_barrier()
    for off, sz in WB:
        pltpu.sync_copy(acc_sh.at[pl.ds(base + off, sz)], rows0.at[pl.ds(0, sz)])
        pltpu.sync_copy(rows0.at[pl.ds(0, sz)], out_hbm.at[c, pl.ds(base + off, sz)])


def _prop_call(dh, ta, tb, src2d, dst2d, ew2d, zrows):
    return pl.kernel(
        functools.partial(_prop_body, dh),
        out_type=jax.ShapeDtypeStruct((NC, N, dh), jnp.float32),
        mesh=_mesh(),
        scratch_types=[
            pltpu.VMEM((CPT, CHUNK), jnp.int32),
            pltpu.VMEM((CPT, CHUNK), jnp.int32),
            pltpu.VMEM((CPT, CHUNK), jnp.float32),
            pltpu.VMEM((CHUNK, dh), jnp.float32),
            pltpu.VMEM((CHUNK, dh), jnp.float32),
            pltpu.VMEM_SHARED((N, dh), jnp.float32),
            pltpu.SemaphoreType.DMA,
            pltpu.SemaphoreType.DMA,
        ],
    )(ta, tb, src2d, dst2d, ew2d, zrows)


R = 2000


def _tc_a(x_ref, w1_ref, d0_ref, d1_ref, ga_ref, gb_ref, dis_ref):
    deg = 1.0 + d0_ref[...] + d1_ref[...]
    dis = lax.rsqrt(deg)
    h = jnp.dot(x_ref[...], w1_ref[...], preferred_element_type=jnp.float32)
    g = dis * h
    ga_ref[...] = g[:, :128]
    gb_ref[...] = g[:, 128:]
    dis_ref[...] = dis


def _tc_a_call(x, W1, d0, d1):
    grid = (N // R,)
    return pl.pallas_call(
        _tc_a,
        grid=grid,
        in_specs=[
            pl.BlockSpec((R, D_IN), lambda i: (i, 0)),
            pl.BlockSpec((D_IN, D_HID), lambda i: (0, 0)),
            pl.BlockSpec((R, 1), lambda i: (i, 0)),
            pl.BlockSpec((R, 1), lambda i: (i, 0)),
        ],
        out_specs=[
            pl.BlockSpec((R, 128), lambda i: (i, 0)),
            pl.BlockSpec((R, 128), lambda i: (i, 0)),
            pl.BlockSpec((R, 1), lambda i: (i, 0)),
        ],
        out_shape=[
            jax.ShapeDtypeStruct((N, 128), jnp.float32),
            jax.ShapeDtypeStruct((N, 128), jnp.float32),
            jax.ShapeDtypeStruct((N, 1), jnp.float32),
        ],
    )(x, W1, d0, d1)


def _tc_b(a1a_ref, a1b_ref, g1a_ref, g1b_ref, dis_ref, b1_ref, w2_ref,
          g2a_ref, g2b_ref):
    dis = dis_ref[...]
    b1 = b1_ref[...]
    w2 = w2_ref[...]
    ha = jnp.maximum(dis * (a1a_ref[...] + g1a_ref[...]) + b1[:, :128], 0.0)
    hb = jnp.maximum(dis * (a1b_ref[...] + g1b_ref[...]) + b1[:, 128:], 0.0)
    h2 = (jnp.dot(ha, w2[:128, :], preferred_element_type=jnp.float32)
          + jnp.dot(hb, w2[128:, :], preferred_element_type=jnp.float32))
    g2 = dis * h2
    g2a_ref[...] = g2[:, :32]
    g2b_ref[...] = g2[:, 32:]


def _tc_b_call(a1a, a1b, g1a, g1b, dis, b1, W2):
    grid = (N // R,)
    blk = lambda w: pl.BlockSpec((R, w), lambda i: (i, 0))
    return pl.pallas_call(
        _tc_b,
        grid=grid,
        in_specs=[
            blk(128), blk(128), blk(128), blk(128), blk(1),
            pl.BlockSpec((1, D_HID), lambda i: (0, 0)),
            pl.BlockSpec((D_HID, N_CLASS), lambda i: (0, 0)),
        ],
        out_specs=[blk(32), blk(32)],
        out_shape=[
            jax.ShapeDtypeStruct((N, 32), jnp.float32),
            jax.ShapeDtypeStruct((N, 32), jnp.float32),
        ],
    )(a1a, a1b, g1a, g1b, dis, b1, W2)


def _tc_c(a2a_ref, a2b_ref, g2a_ref, g2b_ref, dis_ref, b2_ref, out_ref):
    dis = dis_ref[...]
    b2 = b2_ref[...]
    za = dis * (a2a_ref[...] + g2a_ref[...])
    zb = dis * (a2b_ref[...] + g2b_ref[...])
    z = jnp.concatenate([za, zb], axis=1) + b2
    m = jnp.max(z, axis=1, keepdims=True)
    lse = jnp.log(jnp.sum(jnp.exp(z - m), axis=1, keepdims=True)) + m
    out_ref[...] = z - lse


def _tc_c_call(a2a, a2b, g2a, g2b, dis, b2):
    grid = (N // R,)
    blk = lambda w: pl.BlockSpec((R, w), lambda i: (i, 0))
    return pl.pallas_call(
        _tc_c,
        grid=grid,
        in_specs=[
            blk(32), blk(32), blk(32), blk(32), blk(1),
            pl.BlockSpec((1, N_CLASS), lambda i: (0, 0)),
        ],
        out_specs=blk(N_CLASS),
        out_shape=jax.ShapeDtypeStruct((N, N_CLASS), jnp.float32),
    )(a2a, a2b, g2a, g2b, dis, b2)


@jax.jit
def kernel(x, edge_index, edge_attr, W1, b1, W2, b2):
    pad = EPAD - E
    zi = jnp.zeros((pad,), jnp.int32)
    src2d = jnp.concatenate([edge_index[0], zi]).reshape(EROWS, CHUNK)
    dst2d = jnp.concatenate([edge_index[1], zi]).reshape(EROWS, CHUNK)
    ew2d = jnp.concatenate(
        [edge_attr, jnp.zeros((pad,), jnp.float32)]).reshape(EROWS, CHUNK)
    ew3d = ew2d[..., None]

    zcol = jnp.zeros((CHUNK, 1), jnp.float32)
    z128 = jnp.zeros((CHUNK, 128), jnp.float32)
    z32 = jnp.zeros((CHUNK, 32), jnp.float32)

    deg2 = _deg_call(dst2d, ew3d, zcol)
    ga, gb, dis = _tc_a_call(x, W1, deg2[0], deg2[1])
    acc1 = _prop_call(128, ga, gb, src2d, dst2d, ew2d, z128)
    g2a, g2b = _tc_b_call(acc1[0], acc1[1], ga, gb, dis,
                          b1.reshape(1, D_HID), W2)
    acc2 = _prop_call(32, g2a, g2b, src2d, dst2d, ew2d, z32)
    return _tc_c_call(acc2[0], acc2[1], g2a, g2b, dis,
                      b2.reshape(1, N_CLASS))


# speedup vs baseline: 6.3612x; 6.3612x over previous
import functools

import jax
import jax.numpy as jnp
from jax import lax
from jax.experimental import pallas as pl
from jax.experimental.pallas import tpu as pltpu
from jax.experimental.pallas import tpu_sc as plsc

N = 10000
NP = 10240
E = 160000
D_IN = 256
D_HID = 256
N_CLASS = 64

NC = 2
NS = 16
CHUNK = 128
CPT = 80
PH = 4
CPP = CPT // PH
EPAD = NS * CPT * CHUNK
EROWS = EPAD // CHUNK
NPT = NP // NS
WB = ((0, 128), (128, 128), (256, 128), (384, 128), (512, 128))

_mesh = lambda: plsc.VectorSubcoreMesh(core_axis_name="c", subcore_axis_name="s")


def _splat(v16, lane):
    idx = jnp.full((16,), lane, dtype=jnp.int32)
    return v16.at[idx].get(mode="promise_in_bounds")


def _prop_body(dh, ta, tb, src_hbm, dst16_hbm, ew_hbm, z_hbm, out_hbm,
               src_buf, dst_buf, ew_buf, idx_tmp, rows0, rows1, acc_sh,
               sem0, sem1):
    c = lax.axis_index("c")
    s = lax.axis_index("s")
    base = s * NPT
    for off, sz in WB:
        pltpu.sync_copy(z_hbm.at[pl.ds(0, sz)], acc_sh.at[pl.ds(base + off, sz)])
    plsc.subcore_barrier()

    row0 = s * CPT

    iota = lax.iota(jnp.int32, 16)
    shift = jnp.maximum(iota - 1, 0)

    def scale(j, rows):
        def grp(gi, carry):
            w16 = ew_buf[j, pl.ds(gi * 16, 16)]
            for l in range(16):
                wl = _splat(w16, l)
                e = gi * 16 + l
                for v in range(dh // 16):
                    sl = pl.ds(v * 16, 16)
                    rows[e, sl] = rows[e, sl] * wl
            return carry
        lax.fori_loop(0, CHUNK // 16, grp, 0)

    def scatter(j, rows):
        def sg(g, carry):
            d16 = dst_buf[j * 8 + g]
            srt, perm = plsc.sort_key_val(d16, iota)
            prev = srt.at[shift].get(mode="promise_in_bounds")
            run_start = (srt != prev) | (iota == 0)
            q0 = plsc.cummax(jnp.where(run_start, iota, 0))
            first_orig = perm.at[q0].get(mode="promise_in_bounds")
            _, f = plsc.sort_key_val(perm, first_orig)
            is_dup = f != iota
            any_dup = jnp.max(is_dup.astype(jnp.int32))
            e0 = g * 16

            @pl.when(any_dup > 0)
            def _():
                def lane(m, c2):
                    msk = iota == m
                    dup_m = jnp.max(jnp.where(msk, is_dup.astype(jnp.int32), 0))
                    f_m = jnp.max(jnp.where(msk, f, 0))

                    @pl.when(dup_m > 0)
                    def _():
                        for v in range(dh // 16):
                            sl = pl.ds(v * 16, 16)
                            rows[e0 + f_m, sl] = (rows[e0 + f_m, sl]
                                                  + rows[e0 + m, sl])
                    return c2
                lax.fori_loop(0, 16, lane, 0)

            idx_tmp[0] = jnp.where(is_dup, NP + iota, d16)
            pltpu.sync_copy(rows.at[pl.ds(e0, 16)],
                            acc_sh.at[idx_tmp.at[0]], add=True)
            return carry
        lax.fori_loop(0, CHUNK // 16, sg, 0)

    def run(tbl):
        def phase(h, carry):
            p0 = row0 + h * CPP
            pltpu.sync_copy(src_hbm.at[pl.ds(p0, CPP)], src_buf)
            pltpu.sync_copy(dst16_hbm.at[pl.ds(p0 * 8, CPP * 8)], dst_buf)
            pltpu.sync_copy(ew_hbm.at[pl.ds(p0, CPP)], ew_buf)
            pltpu.async_copy(tbl.at[src_buf.at[0]], rows0, sem0)
            pltpu.async_copy(tbl.at[src_buf.at[1]], rows1, sem1)

            def step(j, rows, sem):
                pltpu.make_async_copy(tbl.at[src_buf.at[j]], rows, sem).wait()
                scale(j, rows)
                scatter(j, rows)
                @pl.when(j + 2 < CPP)
                def _():
                    pltpu.async_copy(tbl.at[src_buf.at[j + 2]], rows, sem)

            def pair(g, c2):
                step(2 * g, rows0, sem0)
                step(2 * g + 1, rows1, sem1)
                return c2

            lax.fori_loop(0, CPP // 2, pair, 0)
            return carry

        lax.fori_loop(0, PH, phase, 0)

    @pl.when(c == 0)
    def _():
        run(ta)

    @pl.when(c == 1)
    def _():
        run(tb)

    plsc.subcore_barrier()
    for off, sz in WB:
        pltpu.sync_copy(acc_sh.at[pl.ds(base + off, sz)], rows0.at[pl.ds(0, sz)])
        pltpu.sync_copy(rows0.at[pl.ds(0, sz)], out_hbm.at[c, pl.ds(base + off, sz)])


def _prop_call(dh, ta, tb, src2d, dst16, ew2d, zrows):
    return pl.kernel(
        functools.partial(_prop_body, dh),
        out_type=jax.ShapeDtypeStruct((NC, NP, dh), jnp.float32),
        mesh=_mesh(),
        compiler_params=pltpu.CompilerParams(use_tc_tiling_on_sc=False,
                                             needs_layout_passes=False),
        scratch_types=[
            pltpu.VMEM((CPP, CHUNK), jnp.int32),
            pltpu.VMEM((CPP * 8, 16), jnp.int32),
            pltpu.VMEM((CPP, CHUNK), jnp.float32),
            pltpu.VMEM((1, 16), jnp.int32),
            pltpu.VMEM((CHUNK, dh), jnp.float32),
            pltpu.VMEM((CHUNK, dh), jnp.float32),
            pltpu.VMEM_SHARED((NP + 16, dh), jnp.float32),
            pltpu.SemaphoreType.DMA,
            pltpu.SemaphoreType.DMA,
        ],
    )(ta, tb, src2d, dst16, ew2d, zrows)


R = 2048


def _tc_a(x_ref, w1_ref, d0_ref, ga_ref, gb_ref, dis_ref):
    deg = 1.0 + d0_ref[...]
    dis = lax.rsqrt(deg)
    h = jnp.dot(x_ref[...], w1_ref[...], preferred_element_type=jnp.float32)
    g = dis * h
    ga_ref[...] = g[:, :128]
    gb_ref[...] = g[:, 128:]
    dis_ref[...] = dis


def _tc_a_call(x, W1, d0):
    grid = (NP // R,)
    return pl.pallas_call(
        _tc_a,
        grid=grid,
        in_specs=[
            pl.BlockSpec((R, D_IN), lambda i: (i, 0)),
            pl.BlockSpec((D_IN, D_HID), lambda i: (0, 0)),
            pl.BlockSpec((R, 1), lambda i: (i, 0)),
        ],
        out_specs=[
            pl.BlockSpec((R, 128), lambda i: (i, 0)),
            pl.BlockSpec((R, 128), lambda i: (i, 0)),
            pl.BlockSpec((R, 1), lambda i: (i, 0)),
        ],
        out_shape=[
            jax.ShapeDtypeStruct((NP, 128), jnp.float32),
            jax.ShapeDtypeStruct((NP, 128), jnp.float32),
            jax.ShapeDtypeStruct((NP, 1), jnp.float32),
        ],
    )(x, W1, d0)


def _tc_b(a1a_ref, a1b_ref, g1a_ref, g1b_ref, dis_ref, b1_ref, w2_ref,
          g2a_ref, g2b_ref):
    dis = dis_ref[...]
    b1 = b1_ref[...]
    w2 = w2_ref[...]
    ha = jnp.maximum(dis * (a1a_ref[...] + g1a_ref[...]) + b1[:, :128], 0.0)
    hb = jnp.maximum(dis * (a1b_ref[...] + g1b_ref[...]) + b1[:, 128:], 0.0)
    h2 = (jnp.dot(ha, w2[:128, :], preferred_element_type=jnp.float32)
          + jnp.dot(hb, w2[128:, :], preferred_element_type=jnp.float32))
    g2 = dis * h2
    g2a_ref[...] = g2[:, :32]
    g2b_ref[...] = g2[:, 32:]


def _tc_b_call(a1a, a1b, g1a, g1b, dis, b1, W2):
    grid = (NP // R,)
    blk = lambda w: pl.BlockSpec((R, w), lambda i: (i, 0))
    return pl.pallas_call(
        _tc_b,
        grid=grid,
        in_specs=[
            blk(128), blk(128), blk(128), blk(128), blk(1),
            pl.BlockSpec((1, D_HID), lambda i: (0, 0)),
            pl.BlockSpec((D_HID, N_CLASS), lambda i: (0, 0)),
        ],
        out_specs=[blk(32), blk(32)],
        out_shape=[
            jax.ShapeDtypeStruct((NP, 32), jnp.float32),
            jax.ShapeDtypeStruct((NP, 32), jnp.float32),
        ],
    )(a1a, a1b, g1a, g1b, dis, b1, W2)


def _tc_c(a2a_ref, a2b_ref, g2a_ref, g2b_ref, dis_ref, b2_ref, out_ref):
    dis = dis_ref[...]
    b2 = b2_ref[...]
    za = dis * (a2a_ref[...] + g2a_ref[...])
    zb = dis * (a2b_ref[...] + g2b_ref[...])
    z = jnp.concatenate([za, zb], axis=1) + b2
    m = jnp.max(z, axis=1, keepdims=True)
    lse = jnp.log(jnp.sum(jnp.exp(z - m), axis=1, keepdims=True)) + m
    out_ref[...] = z - lse


def _tc_c_call(a2a, a2b, g2a, g2b, dis, b2):
    grid = (NP // R,)
    blk = lambda w: pl.BlockSpec((R, w), lambda i: (i, 0))
    return pl.pallas_call(
        _tc_c,
        grid=grid,
        in_specs=[
            blk(32), blk(32), blk(32), blk(32), blk(1),
            pl.BlockSpec((1, N_CLASS), lambda i: (0, 0)),
        ],
        out_specs=blk(N_CLASS),
        out_shape=jax.ShapeDtypeStruct((NP, N_CLASS), jnp.float32),
    )(a2a, a2b, g2a, g2b, dis, b2)


@jax.jit
def kernel(x, edge_index, edge_attr, W1, b1, W2, b2):
    pad = EPAD - E
    zi = jnp.zeros((pad,), jnp.int32)
    src2d = jnp.concatenate([edge_index[0], zi]).reshape(EROWS, CHUNK)
    dst16 = jnp.concatenate([edge_index[1], zi]).reshape(EPAD // 16, 16)
    ew2d = jnp.concatenate(
        [edge_attr, jnp.zeros((pad,), jnp.float32)]).reshape(EROWS, CHUNK)

    z16 = jnp.zeros((CHUNK, 16), jnp.float32)
    z128 = jnp.zeros((CHUNK, 128), jnp.float32)
    z32 = jnp.zeros((CHUNK, 32), jnp.float32)
    ones16 = jnp.ones((NP, 16), jnp.float32)

    xp = jnp.concatenate([x, jnp.zeros((NP - N, D_IN), jnp.float32)])

    deg2 = _prop_call(16, ones16, ones16, src2d, dst16, ew2d, z16)
    ga, gb, dis = _tc_a_call(xp, W1, deg2[0][:, :1])
    acc1 = _prop_call(128, ga, gb, src2d, dst16, ew2d, z128)
    g2a, g2b = _tc_b_call(acc1[0], acc1[1], ga, gb, dis,
                          b1.reshape(1, D_HID), W2)
    acc2 = _prop_call(32, g2a, g2b, src2d, dst16, ew2d, z32)
    out = _tc_c_call(acc2[0], acc2[1], g2a, g2b, dis,
                     b2.reshape(1, N_CLASS))
    return out[:N]


# speedup vs baseline: 7.3756x; 1.1595x over previous
import functools

import jax
import jax.numpy as jnp
from jax import lax
from jax.experimental import pallas as pl
from jax.experimental.pallas import tpu as pltpu
from jax.experimental.pallas import tpu_sc as plsc

N = 10000
NP = 10240
E = 160000
D_IN = 256
D_HID = 256
N_CLASS = 64

NC = 2
NS = 16
CHUNK = 128
CPT = 80
PH = 4
CPP = CPT // PH
EPAD = NS * CPT * CHUNK
EROWS = EPAD // CHUNK
NPT = NP // NS
WB = ((0, 128), (128, 128), (256, 128), (384, 128), (512, 128))

_mesh = lambda: plsc.VectorSubcoreMesh(core_axis_name="c", subcore_axis_name="s")


def _splat(v16, lane):
    idx = jnp.full((16,), lane, dtype=jnp.int32)
    return v16.at[idx].get(mode="promise_in_bounds")


def _prop_body(dh, ta, tb, src_hbm, dst16_hbm, ew_hbm, z_hbm, out_hbm,
               src_buf, dst_buf, ew_buf, idx_tmp, rows0, rows1, acc_sh,
               sem0, sem1, ssem0, ssem1):
    c = lax.axis_index("c")
    s = lax.axis_index("s")
    base = s * NPT
    for off, sz in WB:
        pltpu.sync_copy(z_hbm.at[pl.ds(0, sz)], acc_sh.at[pl.ds(base + off, sz)])
    plsc.subcore_barrier()

    row0 = s * CPT

    iota = lax.iota(jnp.int32, 16)
    shift = jnp.maximum(iota - 1, 0)

    def scale(j, rows):
        def grp(gi, carry):
            w16 = ew_buf[j, pl.ds(gi * 16, 16)]
            for l in range(16):
                wl = _splat(w16, l)
                e = gi * 16 + l
                for v in range(dh // 16):
                    sl = pl.ds(v * 16, 16)
                    rows[e, sl] = rows[e, sl] * wl
            return carry
        lax.fori_loop(0, CHUNK // 16, grp, 0)

    def scatter(j, rows, ssem, ibase):
        def sg(g, carry):
            d16 = dst_buf[j * 8 + g]
            srt, perm = plsc.sort_key_val(d16, iota)
            prev = srt.at[shift].get(mode="promise_in_bounds")
            run_start = (srt != prev) | (iota == 0)
            q0 = plsc.cummax(jnp.where(run_start, iota, 0))
            first_orig = perm.at[q0].get(mode="promise_in_bounds")
            _, f = plsc.sort_key_val(perm, first_orig)
            is_dup = f != iota
            any_dup = jnp.max(is_dup.astype(jnp.int32))
            e0 = g * 16

            @pl.when(any_dup > 0)
            def _():
                def lane(m, c2):
                    msk = iota == m
                    dup_m = jnp.max(jnp.where(msk, is_dup.astype(jnp.int32), 0))
                    f_m = jnp.max(jnp.where(msk, f, 0))

                    @pl.when(dup_m > 0)
                    def _():
                        for v in range(dh // 16):
                            sl = pl.ds(v * 16, 16)
                            rows[e0 + f_m, sl] = (rows[e0 + f_m, sl]
                                                  + rows[e0 + m, sl])
                    return c2
                lax.fori_loop(0, 16, lane, 0)

            idx_tmp[ibase + g] = jnp.where(is_dup, NP + iota, d16)
            pltpu.async_copy(rows.at[pl.ds(e0, 16)],
                             acc_sh.at[idx_tmp.at[ibase + g]], ssem, add=True)
            return carry
        lax.fori_loop(0, CHUNK // 16, sg, 0)

        def sdrain(g, carry):
            pltpu.make_async_copy(rows.at[pl.ds(g * 16, 16)],
                                  acc_sh.at[idx_tmp.at[ibase + g]],
                                  ssem).wait()
            return carry
        lax.fori_loop(0, CHUNK // 16, sdrain, 0)

    def run(tbl):
        def phase(h, carry):
            p0 = row0 + h * CPP
            pltpu.sync_copy(src_hbm.at[pl.ds(p0, CPP)], src_buf)
            pltpu.sync_copy(dst16_hbm.at[pl.ds(p0 * 8, CPP * 8)], dst_buf)
            pltpu.sync_copy(ew_hbm.at[pl.ds(p0, CPP)], ew_buf)
            pltpu.async_copy(tbl.at[src_buf.at[0]], rows0, sem0)
            pltpu.async_copy(tbl.at[src_buf.at[1]], rows1, sem1)

            def step(j, rows, sem, ssem, ibase):
                pltpu.make_async_copy(tbl.at[src_buf.at[j]], rows, sem).wait()
                scale(j, rows)
                scatter(j, rows, ssem, ibase)
                @pl.when(j + 2 < CPP)
                def _():
                    pltpu.async_copy(tbl.at[src_buf.at[j + 2]], rows, sem)

            def pair(g, c2):
                step(2 * g, rows0, sem0, ssem0, 0)
                step(2 * g + 1, rows1, sem1, ssem1, 8)
                return c2

            lax.fori_loop(0, CPP // 2, pair, 0)
            return carry

        lax.fori_loop(0, PH, phase, 0)

    @pl.when(c == 0)
    def _():
        run(ta)

    @pl.when(c == 1)
    def _():
        run(tb)

    plsc.subcore_barrier()
    for off, sz in WB:
        pltpu.sync_copy(acc_sh.at[pl.ds(base + off, sz)], rows0.at[pl.ds(0, sz)])
        pltpu.sync_copy(rows0.at[pl.ds(0, sz)], out_hbm.at[c, pl.ds(base + off, sz)])


def _prop_call(dh, ta, tb, src2d, dst16, ew2d, zrows):
    return pl.kernel(
        functools.partial(_prop_body, dh),
        out_type=jax.ShapeDtypeStruct((NC, NP, dh), jnp.float32),
        mesh=_mesh(),
        compiler_params=pltpu.CompilerParams(use_tc_tiling_on_sc=False,
                                             needs_layout_passes=False),
        scratch_types=[
            pltpu.VMEM((CPP, CHUNK), jnp.int32),
            pltpu.VMEM((CPP * 8, 16), jnp.int32),
            pltpu.VMEM((CPP, CHUNK), jnp.float32),
            pltpu.VMEM((16, 16), jnp.int32),
            pltpu.VMEM((CHUNK, dh), jnp.float32),
            pltpu.VMEM((CHUNK, dh), jnp.float32),
            pltpu.VMEM_SHARED((NP + 16, dh), jnp.float32),
            pltpu.SemaphoreType.DMA,
            pltpu.SemaphoreType.DMA,
            pltpu.SemaphoreType.DMA,
            pltpu.SemaphoreType.DMA,
        ],
    )(ta, tb, src2d, dst16, ew2d, zrows)


R = 2048


def _tc_a(x_ref, w1_ref, d0_ref, ga_ref, gb_ref, dis_ref):
    deg = 1.0 + d0_ref[...]
    dis = lax.rsqrt(deg)
    h = jnp.dot(x_ref[...], w1_ref[...], preferred_element_type=jnp.float32)
    g = dis * h
    ga_ref[...] = g[:, :128]
    gb_ref[...] = g[:, 128:]
    dis_ref[...] = dis


def _tc_a_call(x, W1, d0):
    grid = (NP // R,)
    return pl.pallas_call(
        _tc_a,
        grid=grid,
        in_specs=[
            pl.BlockSpec((R, D_IN), lambda i: (i, 0)),
            pl.BlockSpec((D_IN, D_HID), lambda i: (0, 0)),
            pl.BlockSpec((R, 1), lambda i: (i, 0)),
        ],
        out_specs=[
            pl.BlockSpec((R, 128), lambda i: (i, 0)),
            pl.BlockSpec((R, 128), lambda i: (i, 0)),
            pl.BlockSpec((R, 1), lambda i: (i, 0)),
        ],
        out_shape=[
            jax.ShapeDtypeStruct((NP, 128), jnp.float32),
            jax.ShapeDtypeStruct((NP, 128), jnp.float32),
            jax.ShapeDtypeStruct((NP, 1), jnp.float32),
        ],
    )(x, W1, d0)


def _tc_b(a1a_ref, a1b_ref, g1a_ref, g1b_ref, dis_ref, b1_ref, w2_ref,
          g2a_ref, g2b_ref):
    dis = dis_ref[...]
    b1 = b1_ref[...]
    w2 = w2_ref[...]
    ha = jnp.maximum(dis * (a1a_ref[...] + g1a_ref[...]) + b1[:, :128], 0.0)
    hb = jnp.maximum(dis * (a1b_ref[...] + g1b_ref[...]) + b1[:, 128:], 0.0)
    h2 = (jnp.dot(ha, w2[:128, :], preferred_element_type=jnp.float32)
          + jnp.dot(hb, w2[128:, :], preferred_element_type=jnp.float32))
    g2 = dis * h2
    g2a_ref[...] = g2[:, :32]
    g2b_ref[...] = g2[:, 32:]


def _tc_b_call(a1a, a1b, g1a, g1b, dis, b1, W2):
    grid = (NP // R,)
    blk = lambda w: pl.BlockSpec((R, w), lambda i: (i, 0))
    return pl.pallas_call(
        _tc_b,
        grid=grid,
        in_specs=[
            blk(128), blk(128), blk(128), blk(128), blk(1),
            pl.BlockSpec((1, D_HID), lambda i: (0, 0)),
            pl.BlockSpec((D_HID, N_CLASS), lambda i: (0, 0)),
        ],
        out_specs=[blk(32), blk(32)],
        out_shape=[
            jax.ShapeDtypeStruct((NP, 32), jnp.float32),
            jax.ShapeDtypeStruct((NP, 32), jnp.float32),
        ],
    )(a1a, a1b, g1a, g1b, dis, b1, W2)


def _tc_c(a2a_ref, a2b_ref, g2a_ref, g2b_ref, dis_ref, b2_ref, out_ref):
    dis = dis_ref[...]
    b2 = b2_ref[...]
    za = dis * (a2a_ref[...] + g2a_ref[...])
    zb = dis * (a2b_ref[...] + g2b_ref[...])
    z = jnp.concatenate([za, zb], axis=1) + b2
    m = jnp.max(z, axis=1, keepdims=True)
    lse = jnp.log(jnp.sum(jnp.exp(z - m), axis=1, keepdims=True)) + m
    out_ref[...] = z - lse


def _tc_c_call(a2a, a2b, g2a, g2b, dis, b2):
    grid = (NP // R,)
    blk = lambda w: pl.BlockSpec((R, w), lambda i: (i, 0))
    return pl.pallas_call(
        _tc_c,
        grid=grid,
        in_specs=[
            blk(32), blk(32), blk(32), blk(32), blk(1),
            pl.BlockSpec((1, N_CLASS), lambda i: (0, 0)),
        ],
        out_specs=blk(N_CLASS),
        out_shape=jax.ShapeDtypeStruct((NP, N_CLASS), jnp.float32),
    )(a2a, a2b, g2a, g2b, dis, b2)


@jax.jit
def kernel(x, edge_index, edge_attr, W1, b1, W2, b2):
    pad = EPAD - E
    zi = jnp.zeros((pad,), jnp.int32)
    src2d = jnp.concatenate([edge_index[0], zi]).reshape(EROWS, CHUNK)
    dst16 = jnp.concatenate([edge_index[1], zi]).reshape(EPAD // 16, 16)
    ew2d = jnp.concatenate(
        [edge_attr, jnp.zeros((pad,), jnp.float32)]).reshape(EROWS, CHUNK)

    z16 = jnp.zeros((CHUNK, 16), jnp.float32)
    z128 = jnp.zeros((CHUNK, 128), jnp.float32)
    z32 = jnp.zeros((CHUNK, 32), jnp.float32)
    ones16 = jnp.ones((NP, 16), jnp.float32)

    xp = jnp.concatenate([x, jnp.zeros((NP - N, D_IN), jnp.float32)])

    deg2 = _prop_call(16, ones16, ones16, src2d, dst16, ew2d, z16)
    ga, gb, dis = _tc_a_call(xp, W1, deg2[0][:, :1])
    acc1 = _prop_call(128, ga, gb, src2d, dst16, ew2d, z128)
    g2a, g2b = _tc_b_call(acc1[0], acc1[1], ga, gb, dis,
                          b1.reshape(1, D_HID), W2)
    acc2 = _prop_call(32, g2a, g2b, src2d, dst16, ew2d, z32)
    out = _tc_c_call(acc2[0], acc2[1], g2a, g2b, dis,
                     b2.reshape(1, N_CLASS))
    return out[:N]
